# baseline (speedup 1.0000x reference)
import functools

import jax
import jax.numpy as jnp
from jax import lax
from jax.experimental import pallas as pl
from jax.experimental.pallas import tpu as pltpu
from jax.experimental.pallas import tpu_sc as plsc

BOHR = 0.52917721067121
NC, NS = 2, 16
NW = NC * NS
ROW = 128
CR = 18
NSLOT = 3


def _recip(x):
    i = lax.bitcast_convert_type(x, jnp.int32)
    y = lax.bitcast_convert_type(jnp.int32(0x7EF311C3) - i, jnp.float32)
    y = y * (2.0 - x * y)
    y = y * (2.0 - x * y)
    return y


@functools.lru_cache(maxsize=None)
def _make_sc_kernel(N, E):
    assert E % ROW == 0
    R = E // ROW
    base_rows = R // NW
    X = R % NW
    Np = ((N + 1023) // 1024) * 1024
    n_full, rem = divmod(base_rows, CR)
    CE = CR * ROW

    mesh = plsc.VectorSubcoreMesh(
        core_axis_name="c", subcore_axis_name="s",
        num_cores=NC, num_subcores=NS)

    @functools.partial(
        pl.kernel,
        out_type=jax.ShapeDtypeStruct((NW, Np), jnp.float32),
        mesh=mesh,
        compiler_params=pltpu.CompilerParams(use_tc_tiling_on_sc=False,
                                             needs_layout_passes=False),
        scratch_types=[
            pltpu.VMEM((N,), jnp.float32),
            pltpu.VMEM((Np,), jnp.float32),
            pltpu.VMEM((CE,), jnp.int32),
            pltpu.VMEM((CE,), jnp.int32),
            pltpu.VMEM((CE,), jnp.float32),
            pltpu.VMEM((CE,), jnp.float32),
            pltpu.VMEM((CE,), jnp.int32),
            pltpu.VMEM((CE,), jnp.int32),
            pltpu.VMEM((CE,), jnp.float32),
            pltpu.VMEM((CE,), jnp.float32),
            pltpu.VMEM((CE,), jnp.int32),
            pltpu.VMEM((CE,), jnp.int32),
            pltpu.VMEM((CE,), jnp.float32),
            pltpu.VMEM((CE,), jnp.float32),
            pltpu.VMEM_SHARED((Np,), jnp.float32),
            pltpu.SemaphoreType.DMA,
            pltpu.SemaphoreType.DMA,
            pltpu.SemaphoreType.DMA,
        ],
    )
    def sc_kernel(q_hbm, src_hbm, dst_hbm, dist_hbm, sw_hbm, out_hbm,
                  q_v, accum_v,
                  src0, dst0, dist0, sw0, src1, dst1, dist1, sw1,
                  src2, dst2, dist2, sw2,
                  q_s, sem0, sem1, sem2):
        cid = lax.axis_index("c")
        sid = lax.axis_index("s")
        wid = sid * NC + cid

        slots = ((src0, dst0, dist0, sw0, sem0),
                 (src1, dst1, dist1, sw1, sem1),
                 (src2, dst2, dist2, sw2, sem2))

        def stage(slot, r0, nrows):
            ne = nrows * ROW
            e0 = r0 * ROW
            sv, dv, di, sw, sem = slots[slot]
            return [
                pltpu.async_copy(src_hbm.at[pl.ds(e0, ne)], sv.at[pl.ds(0, ne)], sem),
                pltpu.async_copy(dst_hbm.at[pl.ds(e0, ne)], dv.at[pl.ds(0, ne)], sem),
                pltpu.async_copy(dist_hbm.at[pl.ds(e0, ne)], di.at[pl.ds(0, ne)], sem),
                pltpu.async_copy(sw_hbm.at[pl.ds(e0, ne)], sw.at[pl.ds(0, ne)], sem),
            ]

        def compute(slot, nrows):
            sv, dv, di, sw, _ = slots[slot]

            @plsc.parallel_loop(0, nrows * ROW // 16, 1, unroll=4)
            def _compute(j):
                sl = pl.ds(j * 16, 16)
                qd = plsc.load_gather(q_v, [dv[sl]])
                coef = (sw[sl] * BOHR) * _recip(di[sl])
                plsc.addupdate_scatter(accum_v, [sv[sl]], coef * qd)

        row0 = wid * base_rows
        chunks = [(row0 + c * CR, CR) for c in range(n_full)]
        if rem:
            chunks.append((row0 + n_full * CR, rem))
        descs = {c: stage(c, *chunks[c]) for c in range(min(NSLOT - 1, len(chunks)))}

        @pl.when(sid == 0)
        def _stage_q():
            pltpu.sync_copy(q_hbm, q_s.at[pl.ds(0, N)])

        def _zero(i, c):
            accum_v[pl.ds(i * 16, 16)] = jnp.zeros((16,), jnp.float32)
            return c
        lax.fori_loop(0, Np // 16, _zero, 0)
        plsc.subcore_barrier()
        pltpu.sync_copy(q_s.at[pl.ds(0, N)], q_v)

        for c in range(len(chunks)):
            for d in descs.pop(c):
                d.wait()
            nxt = c + NSLOT - 1
            if nxt < len(chunks):
                descs[nxt] = stage(nxt % NSLOT, *chunks[nxt])
            compute(c % NSLOT, chunks[c][1])

        if X:
            @pl.when(wid < X)
            def _extra():
                r0 = NW * base_rows + wid
                for d in stage(0, r0, 1):
                    d.wait()
                compute(0, 1)

        pltpu.sync_copy(accum_v, out_hbm.at[wid])

    return sc_kernel, Np


def _combine_body(q_ref, p_ref, o_ref):
    o_ref[...] = 0.5 * q_ref[...] * jnp.sum(p_ref[...], axis=0)


def kernel(species, charges, edge_src, edge_dst, distances, switch):
    del species
    N = charges.shape[0]
    E = edge_src.shape[0]
    Ep = -(-E // ROW) * ROW
    if Ep != E:
        pad = Ep - E
        edge_src = jnp.pad(edge_src, (0, pad))
        edge_dst = jnp.pad(edge_dst, (0, pad))
        distances = jnp.pad(distances, (0, pad), constant_values=1.0)
        switch = jnp.pad(switch, (0, pad))

    sc_kernel, Np = _make_sc_kernel(N, Ep)
    partial = sc_kernel(
        charges,
        edge_src.astype(jnp.int32),
        edge_dst.astype(jnp.int32),
        distances,
        switch,
    )

    M = Np // 128
    G = next(g for g in (7, 4, 2, 1) if M % g == 0 and (M // g) % 8 == 0)
    RB = M // G
    qp = jnp.pad(charges, (0, Np - N)).reshape(M, 128)
    out = pl.pallas_call(
        _combine_body,
        grid=(G,),
        in_specs=[
            pl.BlockSpec((RB, 128), lambda i: (i, 0)),
            pl.BlockSpec((NW, RB, 128), lambda i: (0, i, 0)),
        ],
        out_specs=pl.BlockSpec((RB, 128), lambda i: (i, 0)),
        out_shape=jax.ShapeDtypeStruct((M, 128), jnp.float32),
    )(qp, partial.reshape(NW, M, 128))
    return out.reshape(-1)[:N]


# speedup vs baseline: 1.0611x; 1.0611x over previous
import functools

import jax
import jax.numpy as jnp
from jax import lax
from jax.experimental import pallas as pl
from jax.experimental.pallas import tpu as pltpu
from jax.experimental.pallas import tpu_sc as plsc

BOHR = 0.52917721067121
NC, NS = 2, 16
NW = NC * NS
ROW = 128
CR = 20
NSLOT = 3


def _recip(x):
    i = lax.bitcast_convert_type(x, jnp.int32)
    y = lax.bitcast_convert_type(jnp.int32(0x7EF311C3) - i, jnp.float32)
    y = y * (2.0 - x * y)
    y = y * (2.0 - x * y)
    return y


@functools.lru_cache(maxsize=None)
def _make_sc_kernel(N, E):
    assert E % ROW == 0
    R = E // ROW
    base_rows = R // NW
    X = R % NW
    Np = ((N + 1023) // 1024) * 1024
    n_full, rem = divmod(base_rows, CR)
    CE = CR * ROW

    mesh = plsc.VectorSubcoreMesh(
        core_axis_name="c", subcore_axis_name="s",
        num_cores=NC, num_subcores=NS)

    @functools.partial(
        pl.kernel,
        out_type=jax.ShapeDtypeStruct((NW, Np), jnp.float32),
        mesh=mesh,
        compiler_params=pltpu.CompilerParams(use_tc_tiling_on_sc=False,
                                             needs_layout_passes=False),
        scratch_types=[
            pltpu.VMEM((N,), jnp.float32),
            pltpu.VMEM((Np,), jnp.float32),
            pltpu.VMEM((CE,), jnp.int32),
            pltpu.VMEM((CE,), jnp.int32),
            pltpu.VMEM((CE,), jnp.float32),
            pltpu.VMEM((CE,), jnp.float32),
            pltpu.VMEM((CE,), jnp.int32),
            pltpu.VMEM((CE,), jnp.int32),
            pltpu.VMEM((CE,), jnp.float32),
            pltpu.VMEM((CE,), jnp.float32),
            pltpu.VMEM((CE,), jnp.int32),
            pltpu.VMEM((CE,), jnp.int32),
            pltpu.VMEM((CE,), jnp.float32),
            pltpu.VMEM((CE,), jnp.float32),
            pltpu.SemaphoreType.DMA,
            pltpu.SemaphoreType.DMA,
            pltpu.SemaphoreType.DMA,
            pltpu.SemaphoreType.DMA,
        ],
    )
    def sc_kernel(q_hbm, src_hbm, dst_hbm, dist_hbm, sw_hbm, out_hbm,
                  q_v, accum_v,
                  src0, dst0, dist0, sw0, src1, dst1, dist1, sw1,
                  src2, dst2, dist2, sw2,
                  sem0, sem1, sem2, qsem):
        cid = lax.axis_index("c")
        sid = lax.axis_index("s")
        wid = sid * NC + cid

        slots = ((src0, dst0, dist0, sw0, sem0),
                 (src1, dst1, dist1, sw1, sem1),
                 (src2, dst2, dist2, sw2, sem2))

        def stage(slot, r0, nrows):
            ne = nrows * ROW
            e0 = r0 * ROW
            sv, dv, di, sw, sem = slots[slot]
            return [
                pltpu.async_copy(src_hbm.at[pl.ds(e0, ne)], sv.at[pl.ds(0, ne)], sem),
                pltpu.async_copy(dst_hbm.at[pl.ds(e0, ne)], dv.at[pl.ds(0, ne)], sem),
                pltpu.async_copy(dist_hbm.at[pl.ds(e0, ne)], di.at[pl.ds(0, ne)], sem),
                pltpu.async_copy(sw_hbm.at[pl.ds(e0, ne)], sw.at[pl.ds(0, ne)], sem),
            ]

        def compute(slot, nrows):
            sv, dv, di, sw, _ = slots[slot]

            @plsc.parallel_loop(0, nrows * ROW // 16, 1, unroll=4)
            def _compute(j):
                sl = pl.ds(j * 16, 16)
                qd = plsc.load_gather(q_v, [dv[sl]])
                coef = (sw[sl] * BOHR) * _recip(di[sl])
                plsc.addupdate_scatter(accum_v, [sv[sl]], coef * qd)

        qdesc = pltpu.async_copy(q_hbm, q_v, qsem)
        row0 = wid * base_rows
        chunks = [(row0 + c * CR, CR) for c in range(n_full)]
        if rem:
            chunks.append((row0 + n_full * CR, rem))
        descs = {c: stage(c, *chunks[c]) for c in range(min(NSLOT - 1, len(chunks)))}

        def _zero(i, c):
            accum_v[pl.ds(i * 16, 16)] = jnp.zeros((16,), jnp.float32)
            return c
        lax.fori_loop(0, Np // 16, _zero, 0)
        qdesc.wait()

        for c in range(len(chunks)):
            for d in descs.pop(c):
                d.wait()
            nxt = c + NSLOT - 1
            if nxt < len(chunks):
                descs[nxt] = stage(nxt % NSLOT, *chunks[nxt])
            compute(c % NSLOT, chunks[c][1])

        if X:
            @pl.when(wid < X)
            def _extra():
                r0 = NW * base_rows + wid
                for d in stage(0, r0, 1):
                    d.wait()
                compute(0, 1)

        pltpu.sync_copy(accum_v, out_hbm.at[wid])

    return sc_kernel, Np


def _combine_body(q_ref, p_ref, o_ref):
    o_ref[...] = 0.5 * q_ref[...] * jnp.sum(p_ref[...], axis=0)


def kernel(species, charges, edge_src, edge_dst, distances, switch):
    del species
    N = charges.shape[0]
    E = edge_src.shape[0]
    Ep = -(-E // ROW) * ROW
    if Ep != E:
        pad = Ep - E
        edge_src = jnp.pad(edge_src, (0, pad))
        edge_dst = jnp.pad(edge_dst, (0, pad))
        distances = jnp.pad(distances, (0, pad), constant_values=1.0)
        switch = jnp.pad(switch, (0, pad))

    sc_kernel, Np = _make_sc_kernel(N, Ep)
    partial = sc_kernel(
        charges,
        edge_src.astype(jnp.int32),
        edge_dst.astype(jnp.int32),
        distances,
        switch,
    )

    M = Np // 128
    G = next(g for g in (7, 4, 2, 1) if M % g == 0 and (M // g) % 8 == 0)
    RB = M // G
    qp = jnp.pad(charges, (0, Np - N)).reshape(M, 128)
    out = pl.pallas_call(
        _combine_body,
        grid=(G,),
        in_specs=[
            pl.BlockSpec((RB, 128), lambda i: (i, 0)),
            pl.BlockSpec((NW, RB, 128), lambda i: (0, i, 0)),
        ],
        out_specs=pl.BlockSpec((RB, 128), lambda i: (i, 0)),
        out_shape=jax.ShapeDtypeStruct((M, 128), jnp.float32),
    )(qp, partial.reshape(NW, M, 128))
    return out.reshape(-1)[:N]
